# baseline (speedup 1.0000x reference)
import functools

import jax
import jax.numpy as jnp
import numpy as np
from jax import lax
from jax.experimental import pallas as pl
from jax.experimental.pallas import tpu as pltpu
from jax.experimental.pallas import tpu_sc as plsc

N = 10000
E = 320000
G = 256
D = 128
RDKIT = 200
HID = 512

NC = 2
NS = 16
NW = NC * NS
EPW = E // NW
KCH = 40
NCHUNK = EPW // KCH
ZR = 40
NZCH = N // ZR

RB = 1000
NBLK = N // RB

_PERM = np.arange(128).reshape(4, 2, 16).transpose(0, 2, 1).reshape(128)


def _prelude_body(x_ref, emb_ref, be_ref, h_ref, t_ref):
    xb = x_ref[...]
    iot = lax.broadcasted_iota(jnp.int32, (RB, 128), 1)
    acc = jnp.zeros((RB, D), jnp.float32)
    for i in range(9):
        oh = (iot == xb[:, i:i + 1]).astype(jnp.float32)
        acc = acc + jnp.dot(oh, emb_ref[i], preferred_element_type=jnp.float32)
    h_ref[...] = acc

    @pl.when(pl.program_id(0) == 0)
    def _():
        r = lax.broadcasted_iota(jnp.int32, (4096, 16), 0)
        c = lax.broadcasted_iota(jnp.int32, (4096, 16), 1)
        t = jnp.dot(((r >> 8) == c).astype(jnp.float32), be_ref[0],
                    preferred_element_type=jnp.float32)
        t = t + jnp.dot((((r >> 4) & 15) == c).astype(jnp.float32), be_ref[1],
                        preferred_element_type=jnp.float32)
        t = t + jnp.dot(((r & 15) == c).astype(jnp.float32), be_ref[2],
                        preferred_element_type=jnp.float32)
        t_ref[...] = t


def _prelude(x, emb_pad, bond_emb):
    return pl.pallas_call(
        _prelude_body,
        grid=(NBLK,),
        in_specs=[
            pl.BlockSpec((RB, 9), lambda i: (i, 0)),
            pl.BlockSpec((9, 128, D), lambda i: (0, 0, 0)),
            pl.BlockSpec((3, 16, D), lambda i: (0, 0, 0)),
        ],
        out_specs=[
            pl.BlockSpec((RB, D), lambda i: (i, 0)),
            pl.BlockSpec((4096, D), lambda i: (0, 0)),
        ],
        out_shape=[
            jax.ShapeDtypeStruct((N, D), jnp.float32),
            jax.ShapeDtypeStruct((4096, D), jnp.float32),
        ],
    )(x, emb_pad, bond_emb)


_sc_mesh = plsc.VectorSubcoreMesh(
    core_axis_name="c", subcore_axis_name="s", num_cores=NC, num_subcores=NS)


HS = 4
NCPAD = -(-NCHUNK // HS) * HS


@functools.partial(
    pl.kernel,
    out_type=jax.ShapeDtypeStruct((NC * N, D), jnp.float32),
    mesh=_sc_mesh,
    scratch_types=[
        [pltpu.VMEM((2 * KCH,), jnp.int32) for _ in range(HS)],
        [pltpu.VMEM((KCH,), jnp.int32) for _ in range(HS)],
        [pltpu.VMEM((2 * KCH, D), jnp.float32) for _ in range(HS)],
        pltpu.VMEM((8, D), jnp.float32),
        pltpu.VMEM_SHARED((N, D), jnp.float32),
        [pltpu.SemaphoreType.DMA for _ in range(HS)],
        [pltpu.SemaphoreType.DMA for _ in range(HS)],
        [pltpu.SemaphoreType.DMA for _ in range(HS)],
        [pltpu.SemaphoreType.DMA for _ in range(HS)],
    ],
)
def _edge_kernel(c_hbm, cidx_hbm, dst_hbm, out_hbm,
                 cidxb, didxb, cbufs, zbuf, aggr,
                 isem, dsem, gsem, ssem):
    cid = lax.axis_index("c")
    sid = lax.axis_index("s")
    wid = sid * NC + cid
    ebase = wid * EPW

    zv = jnp.zeros((16,), jnp.float32)

    @pl.loop(0, 8)
    def _zero_zbuf(r):
        for c8 in range(D // 16):
            zbuf[r, pl.ds(c8 * 16, 16)] = zv

    @pl.loop(sid, NZCH, step=NS)
    def _zero_aggr(j):
        for i in range(ZR // 8):
            pltpu.sync_copy(zbuf, aggr.at[pl.ds(j * ZR + i * 8, 8)])

    plsc.subcore_barrier()

    def c_load(k, s):
        return pltpu.make_async_copy(
            cidx_hbm.at[pl.ds(2 * (ebase + k * KCH), 2 * KCH)],
            cidxb[s], isem[s])

    def d_load(k, s):
        off = ebase + k * KCH
        return pltpu.make_async_copy(dst_hbm.at[pl.ds(off, KCH)],
                                     didxb[s], dsem[s])

    def gather(s):
        return pltpu.make_async_copy(c_hbm.at[cidxb[s]], cbufs[s], gsem[s])

    def scatter(s):
        return pltpu.make_async_copy(cbufs[s].at[pl.ds(0, KCH)],
                                     aggr.at[didxb[s]], ssem[s])

    for k in range(3):
        c_load(k, k).start()
    for k in range(2):
        d_load(k, k).start()
    for k in range(2):
        c_load(k, k).wait()
        gather(k).start()

    @pl.loop(0, NCPAD, step=HS)
    def _chunks(ci):
        for j in range(HS):
            k = ci + j
            s = j
            s2 = (j + 2) % HS
            s3 = (j + 3) % HS

            @pl.when(k >= 2)
            def _(s2=s2):
                scatter(s2).wait()

            @pl.when(k + 2 < NCHUNK)
            def _(k=k, s2=s2):
                d_load(k + 2, s2).start()

            @pl.when(k + 3 < NCHUNK)
            def _(k=k, s3=s3):
                c_load(k + 3, s3).start()

            @pl.when(k + 2 < NCHUNK)
            def _(k=k, s2=s2):
                c_load(k + 2, s2).wait()
                gather(s2).start()

            @pl.when(k < NCHUNK)
            def _(k=k, s=s):
                gather(s).wait()
                cbuf = cbufs[s]

                @pl.loop(0, KCH)
                def _rows(r):
                    for c8 in range(D // 16):
                        sl = pl.ds(c8 * 16, 16)
                        cbuf[r, sl] = jnp.maximum(
                            cbuf[r, sl] + cbuf[KCH + r, sl], 0.0)

                d_load(k, s).wait()
                scatter(s).start(add=True)

    plsc.subcore_barrier()

    @pl.loop(sid, NZCH, step=NS)
    def _writeback(j):
        pltpu.sync_copy(aggr.at[pl.ds(j * ZR, ZR)],
                        out_hbm.at[pl.ds(cid * N + j * ZR, ZR)])


def _update_body(h_ref, a0_ref, a1_ref, wa_ref, ba_ref, wb_ref, bb_ref, o_ref):
    z = h_ref[...] + a0_ref[0] + a1_ref[0]
    y = jnp.maximum(
        jnp.dot(z, wa_ref[...], preferred_element_type=jnp.float32)
        + ba_ref[...], 0.0)
    o_ref[...] = (jnp.dot(y, wb_ref[...], preferred_element_type=jnp.float32)
                  + bb_ref[...])


def _node_update(h, agg, wa, ba, wb, bb):
    return pl.pallas_call(
        _update_body,
        grid=(NBLK,),
        in_specs=[
            pl.BlockSpec((RB, D), lambda i: (i, 0)),
            pl.BlockSpec((1, RB, D), lambda i: (0, i, 0)),
            pl.BlockSpec((1, RB, D), lambda i: (1, i, 0)),
            pl.BlockSpec((D, D), lambda i: (0, 0)),
            pl.BlockSpec((1, D), lambda i: (0, 0)),
            pl.BlockSpec((D, D), lambda i: (0, 0)),
            pl.BlockSpec((1, D), lambda i: (0, 0)),
        ],
        out_specs=pl.BlockSpec((RB, D), lambda i: (i, 0)),
        out_shape=jax.ShapeDtypeStruct((N, D), jnp.float32),
    )(h, agg, agg, wa, ba, wb, bb)


def _update_pool_body(h_ref, a0_ref, a1_ref, wa_ref, ba_ref, wb_ref, bb_ref,
                      bt_ref, rd_ref, w1a_ref, w1b_ref, b1_ref,
                      w2_ref, b2_ref, w3_ref, b3_ref, o_ref, sums, cnts):
    i = pl.program_id(0)

    z = h_ref[...] + a0_ref[0] + a1_ref[0]
    y = jnp.maximum(
        jnp.dot(z, wa_ref[...], preferred_element_type=jnp.float32)
        + ba_ref[...], 0.0)
    h2 = (jnp.dot(y, wb_ref[...], preferred_element_type=jnp.float32)
          + bb_ref[...])

    @pl.when(i == 0)
    def _():
        sums[...] = jnp.zeros((G, D), jnp.float32)
        cnts[...] = jnp.zeros((G, D), jnp.float32)

    b = bt_ref[0]
    oh = (lax.broadcasted_iota(jnp.int32, (G, RB), 0)
          == jnp.broadcast_to(b, (G, RB))).astype(jnp.float32)
    sums[...] += jnp.dot(oh, h2, preferred_element_type=jnp.float32)
    cnts[...] += jnp.dot(oh, jnp.ones((RB, D), jnp.float32),
                         preferred_element_type=jnp.float32)

    @pl.when(i == NBLK - 1)
    def _():
        pooled = sums[...] / jnp.maximum(cnts[...], 1.0)
        z1 = jnp.maximum(
            jnp.dot(pooled, w1a_ref[...], preferred_element_type=jnp.float32)
            + jnp.dot(rd_ref[...], w1b_ref[...],
                      preferred_element_type=jnp.float32)
            + b1_ref[...], 0.0)
        z2 = jnp.maximum(
            jnp.dot(z1, w2_ref[...], preferred_element_type=jnp.float32)
            + b2_ref[...], 0.0)
        o_ref[...] = (jnp.dot(z2, w3_ref[...],
                              preferred_element_type=jnp.float32)
                      + b3_ref[...])


def _update_pool(h, agg, wa, ba, wb, bb, batch3, rdkit,
                 w1a, w1b, b1, w2, b2, w3p, b3p):
    return pl.pallas_call(
        _update_pool_body,
        grid=(NBLK,),
        in_specs=[
            pl.BlockSpec((RB, D), lambda i: (i, 0)),
            pl.BlockSpec((1, RB, D), lambda i: (0, i, 0)),
            pl.BlockSpec((1, RB, D), lambda i: (1, i, 0)),
            pl.BlockSpec((D, D), lambda i: (0, 0)),
            pl.BlockSpec((1, D), lambda i: (0, 0)),
            pl.BlockSpec((D, D), lambda i: (0, 0)),
            pl.BlockSpec((1, D), lambda i: (0, 0)),
            pl.BlockSpec((1, 1, RB), lambda i: (i, 0, 0)),
            pl.BlockSpec((G, RDKIT), lambda i: (0, 0)),
            pl.BlockSpec((D, HID), lambda i: (0, 0)),
            pl.BlockSpec((RDKIT, HID), lambda i: (0, 0)),
            pl.BlockSpec((1, HID), lambda i: (0, 0)),
            pl.BlockSpec((HID, HID // 2), lambda i: (0, 0)),
            pl.BlockSpec((1, HID // 2), lambda i: (0, 0)),
            pl.BlockSpec((HID // 2, 128), lambda i: (0, 0)),
            pl.BlockSpec((1, 128), lambda i: (0, 0)),
        ],
        out_specs=pl.BlockSpec((G, 128), lambda i: (0, 0)),
        out_shape=jax.ShapeDtypeStruct((G, 128), jnp.float32),
        scratch_shapes=[
            pltpu.VMEM((G, D), jnp.float32),
            pltpu.VMEM((G, D), jnp.float32),
        ],
    )(h, agg, agg, wa, ba, wb, bb, batch3, rdkit,
      w1a, w1b, b1, w2, b2, w3p, b3p)


def kernel(x, edge_index, edge_attr, batch, rdkit_feats, atom_emb, bond_emb,
           W1a, b1a, W1b, b1b, W2a, b2a, W2b, b2b,
           M1W, M1b, M2W, M2b, M3W, M3b):
    emb_pad = jnp.pad(atom_emb, ((0, 0), (0, 128 - 100), (0, 0)))
    h, ttab = _prelude(x, emb_pad, bond_emb)

    dst = edge_index[1]
    key = edge_attr[:, 0] * 256 + edge_attr[:, 1] * 16 + edge_attr[:, 2]
    cidx = jnp.concatenate(
        [edge_index[0].reshape(NW, NCHUNK, 1, KCH),
         (key + N).reshape(NW, NCHUNK, 1, KCH)], axis=2).reshape(-1)

    agg = _edge_kernel(jnp.concatenate([h, ttab]), cidx, dst)
    agg = agg.reshape(2, N, D)
    h = _node_update(h, agg, W1a, b1a[None], W1b, b1b[None])
    agg = _edge_kernel(jnp.concatenate([h, ttab]), cidx, dst)
    agg = agg.reshape(2, N, D)

    batch3 = batch.reshape(NBLK, 1, RB)
    w3p = jnp.pad(M3W, ((0, 0), (0, 127)))
    b3p = jnp.pad(M3b[None], ((0, 0), (0, 127)))
    out_full = _update_pool(h, agg, W2a, b2a[None], W2b, b2b[None],
                            batch3, rdkit_feats,
                            M1W[:D], M1W[D:], M1b[None], M2W, M2b[None],
                            w3p, b3p)
    return out_full[:, :1]


# speedup vs baseline: 1.1168x; 1.1168x over previous
import functools

import jax
import jax.numpy as jnp
import numpy as np
from jax import lax
from jax.experimental import pallas as pl
from jax.experimental.pallas import tpu as pltpu
from jax.experimental.pallas import tpu_sc as plsc

N = 10000
E = 320000
G = 256
D = 128
RDKIT = 200
HID = 512

NC = 2
NS = 16
NW = NC * NS
EPW = E // NW
KCH = 40
NCHUNK = EPW // KCH
ZR = 40
NZCH = N // ZR

RB = 1000
NBLK = N // RB

_PERM = np.arange(128).reshape(4, 2, 16).transpose(0, 2, 1).reshape(128)


def _prelude_body(x_ref, emb_ref, be_ref, h_ref, t_ref):
    xb = x_ref[...]
    iot = lax.broadcasted_iota(jnp.int32, (RB, 128), 1)
    acc = jnp.zeros((RB, D), jnp.float32)
    for i in range(9):
        oh = (iot == xb[:, i:i + 1]).astype(jnp.float32)
        acc = acc + jnp.dot(oh, emb_ref[i], preferred_element_type=jnp.float32)
    h_ref[...] = acc

    @pl.when(pl.program_id(0) == 0)
    def _():
        r = lax.broadcasted_iota(jnp.int32, (4096, 16), 0)
        c = lax.broadcasted_iota(jnp.int32, (4096, 16), 1)
        t = jnp.dot(((r >> 8) == c).astype(jnp.float32), be_ref[0],
                    preferred_element_type=jnp.float32)
        t = t + jnp.dot((((r >> 4) & 15) == c).astype(jnp.float32), be_ref[1],
                        preferred_element_type=jnp.float32)
        t = t + jnp.dot(((r & 15) == c).astype(jnp.float32), be_ref[2],
                        preferred_element_type=jnp.float32)
        t_ref[...] = t


def _prelude(x, emb_pad, bond_emb):
    return pl.pallas_call(
        _prelude_body,
        grid=(NBLK,),
        in_specs=[
            pl.BlockSpec((RB, 9), lambda i: (i, 0)),
            pl.BlockSpec((9, 128, D), lambda i: (0, 0, 0)),
            pl.BlockSpec((3, 16, D), lambda i: (0, 0, 0)),
        ],
        out_specs=[
            pl.BlockSpec((RB, D), lambda i: (i, 0)),
            pl.BlockSpec((4096, D), lambda i: (0, 0)),
        ],
        out_shape=[
            jax.ShapeDtypeStruct((N, D), jnp.float32),
            jax.ShapeDtypeStruct((4096, D), jnp.float32),
        ],
    )(x, emb_pad, bond_emb)


_sc_mesh = plsc.VectorSubcoreMesh(
    core_axis_name="c", subcore_axis_name="s", num_cores=NC, num_subcores=NS)


HS = 4
NCPAD = -(-NCHUNK // HS) * HS


@functools.partial(
    pl.kernel,
    out_type=jax.ShapeDtypeStruct((NC * N, D), jnp.float32),
    mesh=_sc_mesh,
    scratch_types=[
        [pltpu.VMEM((KCH,), jnp.int32) for _ in range(HS)],
        [pltpu.VMEM((KCH,), jnp.int32) for _ in range(HS)],
        [pltpu.VMEM((KCH,), jnp.int32) for _ in range(HS)],
        [pltpu.VMEM((KCH, D), jnp.float32) for _ in range(HS)],
        [pltpu.VMEM((KCH, D), jnp.float32) for _ in range(HS)],
        pltpu.VMEM((8, D), jnp.float32),
        pltpu.VMEM_SHARED((N, D), jnp.float32),
        [pltpu.SemaphoreType.DMA for _ in range(HS)],
        [pltpu.SemaphoreType.DMA for _ in range(HS)],
        [pltpu.SemaphoreType.DMA for _ in range(HS)],
        [pltpu.SemaphoreType.DMA for _ in range(HS)],
        [pltpu.SemaphoreType.DMA for _ in range(HS)],
    ],
)
def _edge_kernel(h_hbm, t_hbm, src_hbm, key_hbm, dst_hbm, out_hbm,
                 sidxb, kidxb, didxb, hbufs, tbufs, zbuf, aggr,
                 isem, dsem, gsh, gst, ssem):
    cid = lax.axis_index("c")
    sid = lax.axis_index("s")
    wid = sid * NC + cid
    ebase = wid * EPW

    zv = jnp.zeros((16,), jnp.float32)

    @pl.loop(0, 8)
    def _zero_zbuf(r):
        for c8 in range(D // 16):
            zbuf[r, pl.ds(c8 * 16, 16)] = zv

    @pl.loop(sid, NZCH, step=NS)
    def _zero_aggr(j):
        for i in range(ZR // 8):
            pltpu.sync_copy(zbuf, aggr.at[pl.ds(j * ZR + i * 8, 8)])

    plsc.subcore_barrier()

    def sk_loads(k, s):
        off = ebase + k * KCH
        return (pltpu.make_async_copy(src_hbm.at[pl.ds(off, KCH)],
                                      sidxb[s], isem[s]),
                pltpu.make_async_copy(key_hbm.at[pl.ds(off, KCH)],
                                      kidxb[s], isem[s]))

    def d_load(k, s):
        off = ebase + k * KCH
        return pltpu.make_async_copy(dst_hbm.at[pl.ds(off, KCH)],
                                     didxb[s], dsem[s])

    def h_gather(s):
        return pltpu.make_async_copy(h_hbm.at[sidxb[s]], hbufs[s], gsh[s])

    def t_gather(s):
        return pltpu.make_async_copy(t_hbm.at[kidxb[s]], tbufs[s], gst[s])

    def scatter(s):
        return pltpu.make_async_copy(hbufs[s], aggr.at[didxb[s]], ssem[s])

    for k in range(3):
        for cp in sk_loads(k, k):
            cp.start()
    for k in range(2):
        d_load(k, k).start()
    for k in range(2):
        for cp in sk_loads(k, k):
            cp.wait()
        h_gather(k).start()
        t_gather(k).start()

    @pl.loop(0, NCPAD, step=HS)
    def _chunks(ci):
        for j in range(HS):
            k = ci + j
            s = j
            s2 = (j + 2) % HS
            s3 = (j + 3) % HS

            @pl.when(k >= 2)
            def _(s2=s2):
                scatter(s2).wait()

            @pl.when(k + 2 < NCHUNK)
            def _(k=k, s2=s2):
                d_load(k + 2, s2).start()

            @pl.when(k + 3 < NCHUNK)
            def _(k=k, s3=s3):
                for cp in sk_loads(k + 3, s3):
                    cp.start()

            @pl.when(k + 2 < NCHUNK)
            def _(k=k, s2=s2):
                for cp in sk_loads(k + 2, s2):
                    cp.wait()
                h_gather(s2).start()
                t_gather(s2).start()

            @pl.when(k < NCHUNK)
            def _(k=k, s=s):
                h_gather(s).wait()
                t_gather(s).wait()
                hbuf, tbuf = hbufs[s], tbufs[s]

                @pl.loop(0, KCH)
                def _rows(r):
                    for c8 in range(D // 16):
                        sl = pl.ds(c8 * 16, 16)
                        hbuf[r, sl] = jnp.maximum(hbuf[r, sl] + tbuf[r, sl],
                                                  0.0)

                d_load(k, s).wait()
                scatter(s).start(add=True)

    plsc.subcore_barrier()

    @pl.loop(sid, NZCH, step=NS)
    def _writeback(j):
        pltpu.sync_copy(aggr.at[pl.ds(j * ZR, ZR)],
                        out_hbm.at[pl.ds(cid * N + j * ZR, ZR)])


def _update_body(h_ref, a0_ref, a1_ref, wa_ref, ba_ref, wb_ref, bb_ref, o_ref):
    z = h_ref[...] + a0_ref[0] + a1_ref[0]
    y = jnp.maximum(
        jnp.dot(z, wa_ref[...], preferred_element_type=jnp.float32)
        + ba_ref[...], 0.0)
    o_ref[...] = (jnp.dot(y, wb_ref[...], preferred_element_type=jnp.float32)
                  + bb_ref[...])


def _node_update(h, agg, wa, ba, wb, bb):
    return pl.pallas_call(
        _update_body,
        grid=(NBLK,),
        in_specs=[
            pl.BlockSpec((RB, D), lambda i: (i, 0)),
            pl.BlockSpec((1, RB, D), lambda i: (0, i, 0)),
            pl.BlockSpec((1, RB, D), lambda i: (1, i, 0)),
            pl.BlockSpec((D, D), lambda i: (0, 0)),
            pl.BlockSpec((1, D), lambda i: (0, 0)),
            pl.BlockSpec((D, D), lambda i: (0, 0)),
            pl.BlockSpec((1, D), lambda i: (0, 0)),
        ],
        out_specs=pl.BlockSpec((RB, D), lambda i: (i, 0)),
        out_shape=jax.ShapeDtypeStruct((N, D), jnp.float32),
    )(h, agg, agg, wa, ba, wb, bb)


def _update_pool_body(h_ref, a0_ref, a1_ref, wa_ref, ba_ref, wb_ref, bb_ref,
                      bt_ref, rd_ref, w1a_ref, w1b_ref, b1_ref,
                      w2_ref, b2_ref, w3_ref, b3_ref, o_ref, sums, cnts):
    i = pl.program_id(0)

    z = h_ref[...] + a0_ref[0] + a1_ref[0]
    y = jnp.maximum(
        jnp.dot(z, wa_ref[...], preferred_element_type=jnp.float32)
        + ba_ref[...], 0.0)
    h2 = (jnp.dot(y, wb_ref[...], preferred_element_type=jnp.float32)
          + bb_ref[...])

    @pl.when(i == 0)
    def _():
        sums[...] = jnp.zeros((G, D), jnp.float32)
        cnts[...] = jnp.zeros((G, D), jnp.float32)

    b = bt_ref[0]
    oh = (lax.broadcasted_iota(jnp.int32, (G, RB), 0)
          == jnp.broadcast_to(b, (G, RB))).astype(jnp.float32)
    sums[...] += jnp.dot(oh, h2, preferred_element_type=jnp.float32)
    cnts[...] += jnp.dot(oh, jnp.ones((RB, D), jnp.float32),
                         preferred_element_type=jnp.float32)

    @pl.when(i == NBLK - 1)
    def _():
        pooled = sums[...] / jnp.maximum(cnts[...], 1.0)
        z1 = jnp.maximum(
            jnp.dot(pooled, w1a_ref[...], preferred_element_type=jnp.float32)
            + jnp.dot(rd_ref[...], w1b_ref[...],
                      preferred_element_type=jnp.float32)
            + b1_ref[...], 0.0)
        z2 = jnp.maximum(
            jnp.dot(z1, w2_ref[...], preferred_element_type=jnp.float32)
            + b2_ref[...], 0.0)
        o_ref[...] = (jnp.dot(z2, w3_ref[...],
                              preferred_element_type=jnp.float32)
                      + b3_ref[...])


def _update_pool(h, agg, wa, ba, wb, bb, batch3, rdkit,
                 w1a, w1b, b1, w2, b2, w3p, b3p):
    return pl.pallas_call(
        _update_pool_body,
        grid=(NBLK,),
        in_specs=[
            pl.BlockSpec((RB, D), lambda i: (i, 0)),
            pl.BlockSpec((1, RB, D), lambda i: (0, i, 0)),
            pl.BlockSpec((1, RB, D), lambda i: (1, i, 0)),
            pl.BlockSpec((D, D), lambda i: (0, 0)),
            pl.BlockSpec((1, D), lambda i: (0, 0)),
            pl.BlockSpec((D, D), lambda i: (0, 0)),
            pl.BlockSpec((1, D), lambda i: (0, 0)),
            pl.BlockSpec((1, 1, RB), lambda i: (i, 0, 0)),
            pl.BlockSpec((G, RDKIT), lambda i: (0, 0)),
            pl.BlockSpec((D, HID), lambda i: (0, 0)),
            pl.BlockSpec((RDKIT, HID), lambda i: (0, 0)),
            pl.BlockSpec((1, HID), lambda i: (0, 0)),
            pl.BlockSpec((HID, HID // 2), lambda i: (0, 0)),
            pl.BlockSpec((1, HID // 2), lambda i: (0, 0)),
            pl.BlockSpec((HID // 2, 128), lambda i: (0, 0)),
            pl.BlockSpec((1, 128), lambda i: (0, 0)),
        ],
        out_specs=pl.BlockSpec((G, 128), lambda i: (0, 0)),
        out_shape=jax.ShapeDtypeStruct((G, 128), jnp.float32),
        scratch_shapes=[
            pltpu.VMEM((G, D), jnp.float32),
            pltpu.VMEM((G, D), jnp.float32),
        ],
    )(h, agg, agg, wa, ba, wb, bb, batch3, rdkit,
      w1a, w1b, b1, w2, b2, w3p, b3p)


def kernel(x, edge_index, edge_attr, batch, rdkit_feats, atom_emb, bond_emb,
           W1a, b1a, W1b, b1b, W2a, b2a, W2b, b2b,
           M1W, M1b, M2W, M2b, M3W, M3b):
    emb_pad = jnp.pad(atom_emb, ((0, 0), (0, 128 - 100), (0, 0)))
    h, ttab = _prelude(x, emb_pad, bond_emb)

    src = edge_index[0]
    dst = edge_index[1]
    key = edge_attr[:, 0] * 256 + edge_attr[:, 1] * 16 + edge_attr[:, 2]

    agg = _edge_kernel(h, ttab, src, key, dst).reshape(2, N, D)
    h = _node_update(h, agg, W1a, b1a[None], W1b, b1b[None])
    agg = _edge_kernel(h, ttab, src, key, dst).reshape(2, N, D)

    batch3 = batch.reshape(NBLK, 1, RB)
    w3p = jnp.pad(M3W, ((0, 0), (0, 127)))
    b3p = jnp.pad(M3b[None], ((0, 0), (0, 127)))
    out_full = _update_pool(h, agg, W2a, b2a[None], W2b, b2b[None],
                            batch3, rdkit_feats,
                            M1W[:D], M1W[D:], M1b[None], M2W, M2b[None],
                            w3p, b3p)
    return out_full[:, :1]


# speedup vs baseline: 1.1339x; 1.0153x over previous
import functools

import jax
import jax.numpy as jnp
import numpy as np
from jax import lax
from jax.experimental import pallas as pl
from jax.experimental.pallas import tpu as pltpu
from jax.experimental.pallas import tpu_sc as plsc

N = 10000
E = 320000
G = 256
D = 128
RDKIT = 200
HID = 512

NC = 2
NS = 16
NW = NC * NS
EPW = E // NW
KCH = 40
NCHUNK = EPW // KCH
ZR = 40
NZCH = N // ZR

RB = 1000
NBLK = N // RB

_PERM = np.arange(128).reshape(4, 2, 16).transpose(0, 2, 1).reshape(128)


def _prelude_body(x_ref, emb_ref, be_ref, h_ref, t_ref):
    xb = x_ref[...]
    iot = lax.broadcasted_iota(jnp.int32, (RB, 128), 1)
    acc = jnp.zeros((RB, D), jnp.float32)
    for i in range(9):
        oh = (iot == xb[:, i:i + 1]).astype(jnp.float32)
        acc = acc + jnp.dot(oh, emb_ref[i], preferred_element_type=jnp.float32)
    h_ref[...] = acc

    @pl.when(pl.program_id(0) == 0)
    def _():
        r = lax.broadcasted_iota(jnp.int32, (4096, 16), 0)
        c = lax.broadcasted_iota(jnp.int32, (4096, 16), 1)
        t = jnp.dot(((r >> 8) == c).astype(jnp.float32), be_ref[0],
                    preferred_element_type=jnp.float32)
        t = t + jnp.dot((((r >> 4) & 15) == c).astype(jnp.float32), be_ref[1],
                        preferred_element_type=jnp.float32)
        t = t + jnp.dot(((r & 15) == c).astype(jnp.float32), be_ref[2],
                        preferred_element_type=jnp.float32)
        t_ref[...] = t


def _prelude(x, emb_pad, bond_emb):
    return pl.pallas_call(
        _prelude_body,
        grid=(NBLK,),
        in_specs=[
            pl.BlockSpec((RB, 9), lambda i: (i, 0)),
            pl.BlockSpec((9, 128, D), lambda i: (0, 0, 0)),
            pl.BlockSpec((3, 16, D), lambda i: (0, 0, 0)),
        ],
        out_specs=[
            pl.BlockSpec((RB, D), lambda i: (i, 0)),
            pl.BlockSpec((4096, D), lambda i: (0, 0)),
        ],
        out_shape=[
            jax.ShapeDtypeStruct((N, D), jnp.float32),
            jax.ShapeDtypeStruct((4096, D), jnp.float32),
        ],
    )(x, emb_pad, bond_emb)


_sc_mesh = plsc.VectorSubcoreMesh(
    core_axis_name="c", subcore_axis_name="s", num_cores=NC, num_subcores=NS)


HS = 4
NCPAD = -(-NCHUNK // HS) * HS


@functools.partial(
    pl.kernel,
    out_type=jax.ShapeDtypeStruct((NC * N, D), jnp.float32),
    mesh=_sc_mesh,
    scratch_types=[
        [pltpu.VMEM((KCH,), jnp.int32) for _ in range(HS)],
        [pltpu.VMEM((KCH,), jnp.int32) for _ in range(HS)],
        [pltpu.VMEM((KCH,), jnp.int32) for _ in range(HS)],
        [pltpu.VMEM((KCH, D), jnp.float32) for _ in range(HS)],
        [pltpu.VMEM((KCH, D), jnp.float32) for _ in range(HS)],
        pltpu.VMEM((8, D), jnp.float32),
        pltpu.VMEM_SHARED((N, D), jnp.float32),
        [pltpu.SemaphoreType.DMA for _ in range(HS)],
        [pltpu.SemaphoreType.DMA for _ in range(HS)],
        [pltpu.SemaphoreType.DMA for _ in range(HS)],
        [pltpu.SemaphoreType.DMA for _ in range(HS)],
        [pltpu.SemaphoreType.DMA for _ in range(HS)],
    ],
)
def _edge_kernel(h_hbm, t_hbm, src_hbm, key_hbm, dst_hbm, out_hbm,
                 sidxb, kidxb, didxb, hbufs, tbufs, zbuf, aggr,
                 isem, dsem, gsh, gst, ssem):
    cid = lax.axis_index("c")
    sid = lax.axis_index("s")
    wid = sid * NC + cid
    ebase = wid * EPW

    zv = jnp.zeros((16,), jnp.float32)

    @pl.loop(0, 8)
    def _zero_zbuf(r):
        for c8 in range(D // 16):
            zbuf[r, pl.ds(c8 * 16, 16)] = zv

    @pl.loop(sid, NZCH, step=NS)
    def _zero_aggr(j):
        zcps = [pltpu.make_async_copy(zbuf, aggr.at[pl.ds(j * ZR + i * 8, 8)],
                                      isem[0]) for i in range(ZR // 8)]
        for cp in zcps:
            cp.start()
        for cp in zcps:
            cp.wait()

    plsc.subcore_barrier()

    def sk_loads(k, s):
        off = ebase + k * KCH
        return (pltpu.make_async_copy(src_hbm.at[pl.ds(off, KCH)],
                                      sidxb[s], isem[s]),
                pltpu.make_async_copy(key_hbm.at[pl.ds(off, KCH)],
                                      kidxb[s], isem[s]))

    def d_load(k, s):
        off = ebase + k * KCH
        return pltpu.make_async_copy(dst_hbm.at[pl.ds(off, KCH)],
                                     didxb[s], dsem[s])

    def h_gather(s):
        return pltpu.make_async_copy(h_hbm.at[sidxb[s]], hbufs[s], gsh[s])

    def t_gather(s):
        return pltpu.make_async_copy(t_hbm.at[kidxb[s]], tbufs[s], gst[s])

    def scatter(s):
        return pltpu.make_async_copy(hbufs[s], aggr.at[didxb[s]], ssem[s])

    for k in range(3):
        for cp in sk_loads(k, k):
            cp.start()
    for k in range(2):
        d_load(k, k).start()
    for k in range(2):
        for cp in sk_loads(k, k):
            cp.wait()
        h_gather(k).start()
        t_gather(k).start()

    @pl.loop(0, NCPAD, step=HS)
    def _chunks(ci):
        for j in range(HS):
            k = ci + j
            s = j
            s2 = (j + 2) % HS
            s3 = (j + 3) % HS

            @pl.when(k >= 2)
            def _(s2=s2):
                scatter(s2).wait()

            @pl.when(k + 2 < NCHUNK)
            def _(k=k, s2=s2):
                d_load(k + 2, s2).start()

            @pl.when(k + 3 < NCHUNK)
            def _(k=k, s3=s3):
                for cp in sk_loads(k + 3, s3):
                    cp.start()

            @pl.when(k + 2 < NCHUNK)
            def _(k=k, s2=s2):
                for cp in sk_loads(k + 2, s2):
                    cp.wait()
                h_gather(s2).start()
                t_gather(s2).start()

            @pl.when(k < NCHUNK)
            def _(k=k, s=s):
                h_gather(s).wait()
                t_gather(s).wait()
                hbuf, tbuf = hbufs[s], tbufs[s]

                @pl.loop(0, KCH)
                def _rows(r):
                    for c8 in range(D // 16):
                        sl = pl.ds(c8 * 16, 16)
                        hbuf[r, sl] = jnp.maximum(hbuf[r, sl] + tbuf[r, sl],
                                                  0.0)

                d_load(k, s).wait()
                scatter(s).start(add=True)

    plsc.subcore_barrier()

    @pl.loop(sid, NZCH, step=NS)
    def _writeback(j):
        pltpu.sync_copy(aggr.at[pl.ds(j * ZR, ZR)],
                        out_hbm.at[pl.ds(cid * N + j * ZR, ZR)])


def _update_body(h_ref, a0_ref, a1_ref, wa_ref, ba_ref, wb_ref, bb_ref, o_ref):
    z = h_ref[...] + a0_ref[0] + a1_ref[0]
    y = jnp.maximum(
        jnp.dot(z, wa_ref[...], preferred_element_type=jnp.float32)
        + ba_ref[...], 0.0)
    o_ref[...] = (jnp.dot(y, wb_ref[...], preferred_element_type=jnp.float32)
                  + bb_ref[...])


def _node_update(h, agg, wa, ba, wb, bb):
    return pl.pallas_call(
        _update_body,
        grid=(NBLK,),
        in_specs=[
            pl.BlockSpec((RB, D), lambda i: (i, 0)),
            pl.BlockSpec((1, RB, D), lambda i: (0, i, 0)),
            pl.BlockSpec((1, RB, D), lambda i: (1, i, 0)),
            pl.BlockSpec((D, D), lambda i: (0, 0)),
            pl.BlockSpec((1, D), lambda i: (0, 0)),
            pl.BlockSpec((D, D), lambda i: (0, 0)),
            pl.BlockSpec((1, D), lambda i: (0, 0)),
        ],
        out_specs=pl.BlockSpec((RB, D), lambda i: (i, 0)),
        out_shape=jax.ShapeDtypeStruct((N, D), jnp.float32),
    )(h, agg, agg, wa, ba, wb, bb)


def _update_pool_body(h_ref, a0_ref, a1_ref, wa_ref, ba_ref, wb_ref, bb_ref,
                      bt_ref, rd_ref, w1a_ref, w1b_ref, b1_ref,
                      w2_ref, b2_ref, w3_ref, b3_ref, o_ref, sums, cnts):
    i = pl.program_id(0)

    z = h_ref[...] + a0_ref[0] + a1_ref[0]
    y = jnp.maximum(
        jnp.dot(z, wa_ref[...], preferred_element_type=jnp.float32)
        + ba_ref[...], 0.0)
    h2 = (jnp.dot(y, wb_ref[...], preferred_element_type=jnp.float32)
          + bb_ref[...])

    @pl.when(i == 0)
    def _():
        sums[...] = jnp.zeros((G, D), jnp.float32)
        cnts[...] = jnp.zeros((G, D), jnp.float32)

    b = bt_ref[0]
    oh = (lax.broadcasted_iota(jnp.int32, (G, RB), 0)
          == jnp.broadcast_to(b, (G, RB))).astype(jnp.float32)
    sums[...] += jnp.dot(oh, h2, preferred_element_type=jnp.float32)
    cnts[...] += jnp.dot(oh, jnp.ones((RB, D), jnp.float32),
                         preferred_element_type=jnp.float32)

    @pl.when(i == NBLK - 1)
    def _():
        pooled = sums[...] / jnp.maximum(cnts[...], 1.0)
        z1 = jnp.maximum(
            jnp.dot(pooled, w1a_ref[...], preferred_element_type=jnp.float32)
            + jnp.dot(rd_ref[...], w1b_ref[...],
                      preferred_element_type=jnp.float32)
            + b1_ref[...], 0.0)
        z2 = jnp.maximum(
            jnp.dot(z1, w2_ref[...], preferred_element_type=jnp.float32)
            + b2_ref[...], 0.0)
        o_ref[...] = (jnp.dot(z2, w3_ref[...],
                              preferred_element_type=jnp.float32)
                      + b3_ref[...])


def _update_pool(h, agg, wa, ba, wb, bb, batch3, rdkit,
                 w1a, w1b, b1, w2, b2, w3p, b3p):
    return pl.pallas_call(
        _update_pool_body,
        grid=(NBLK,),
        in_specs=[
            pl.BlockSpec((RB, D), lambda i: (i, 0)),
            pl.BlockSpec((1, RB, D), lambda i: (0, i, 0)),
            pl.BlockSpec((1, RB, D), lambda i: (1, i, 0)),
            pl.BlockSpec((D, D), lambda i: (0, 0)),
            pl.BlockSpec((1, D), lambda i: (0, 0)),
            pl.BlockSpec((D, D), lambda i: (0, 0)),
            pl.BlockSpec((1, D), lambda i: (0, 0)),
            pl.BlockSpec((1, 1, RB), lambda i: (i, 0, 0)),
            pl.BlockSpec((G, RDKIT), lambda i: (0, 0)),
            pl.BlockSpec((D, HID), lambda i: (0, 0)),
            pl.BlockSpec((RDKIT, HID), lambda i: (0, 0)),
            pl.BlockSpec((1, HID), lambda i: (0, 0)),
            pl.BlockSpec((HID, HID // 2), lambda i: (0, 0)),
            pl.BlockSpec((1, HID // 2), lambda i: (0, 0)),
            pl.BlockSpec((HID // 2, 128), lambda i: (0, 0)),
            pl.BlockSpec((1, 128), lambda i: (0, 0)),
        ],
        out_specs=pl.BlockSpec((G, 128), lambda i: (0, 0)),
        out_shape=jax.ShapeDtypeStruct((G, 128), jnp.float32),
        scratch_shapes=[
            pltpu.VMEM((G, D), jnp.float32),
            pltpu.VMEM((G, D), jnp.float32),
        ],
    )(h, agg, agg, wa, ba, wb, bb, batch3, rdkit,
      w1a, w1b, b1, w2, b2, w3p, b3p)


def kernel(x, edge_index, edge_attr, batch, rdkit_feats, atom_emb, bond_emb,
           W1a, b1a, W1b, b1b, W2a, b2a, W2b, b2b,
           M1W, M1b, M2W, M2b, M3W, M3b):
    emb_pad = jnp.pad(atom_emb, ((0, 0), (0, 128 - 100), (0, 0)))
    h, ttab = _prelude(x, emb_pad, bond_emb)

    src = edge_index[0]
    dst = edge_index[1]
    key = edge_attr[:, 0] * 256 + edge_attr[:, 1] * 16 + edge_attr[:, 2]

    agg = _edge_kernel(h, ttab, src, key, dst).reshape(2, N, D)
    h = _node_update(h, agg, W1a, b1a[None], W1b, b1b[None])
    agg = _edge_kernel(h, ttab, src, key, dst).reshape(2, N, D)

    batch3 = batch.reshape(NBLK, 1, RB)
    w3p = jnp.pad(M3W, ((0, 0), (0, 127)))
    b3p = jnp.pad(M3b[None], ((0, 0), (0, 127)))
    out_full = _update_pool(h, agg, W2a, b2a[None], W2b, b2b[None],
                            batch3, rdkit_feats,
                            M1W[:D], M1W[D:], M1b[None], M2W, M2b[None],
                            w3p, b3p)
    return out_full[:, :1]


# speedup vs baseline: 1.1552x; 1.0188x over previous
import functools

import jax
import jax.numpy as jnp
import numpy as np
from jax import lax
from jax.experimental import pallas as pl
from jax.experimental.pallas import tpu as pltpu
from jax.experimental.pallas import tpu_sc as plsc

N = 10000
E = 320000
G = 256
D = 128
RDKIT = 200
HID = 512

NC = 2
NS = 16
NW = NC * NS
EPW = E // NW
KCH = 40
NCHUNK = EPW // KCH
ZR = 40
NZCH = N // ZR

RB = 1000
NBLK = N // RB

_PERM = np.arange(128).reshape(4, 2, 16).transpose(0, 2, 1).reshape(128)


def _prelude_body(x_ref, emb_ref, be_ref, h_ref, t_ref):
    xb = x_ref[...]
    iot = lax.broadcasted_iota(jnp.int32, (RB, 128), 1)
    acc = jnp.zeros((RB, D), jnp.float32)
    for i in range(9):
        oh = (iot == xb[:, i:i + 1]).astype(jnp.float32)
        acc = acc + jnp.dot(oh, emb_ref[i], preferred_element_type=jnp.float32)
    h_ref[...] = acc

    @pl.when(pl.program_id(0) == 0)
    def _():
        r = lax.broadcasted_iota(jnp.int32, (4096, 16), 0)
        c = lax.broadcasted_iota(jnp.int32, (4096, 16), 1)
        t = jnp.dot(((r >> 8) == c).astype(jnp.float32), be_ref[0],
                    preferred_element_type=jnp.float32)
        t = t + jnp.dot((((r >> 4) & 15) == c).astype(jnp.float32), be_ref[1],
                        preferred_element_type=jnp.float32)
        t = t + jnp.dot(((r & 15) == c).astype(jnp.float32), be_ref[2],
                        preferred_element_type=jnp.float32)
        t_ref[...] = t


def _prelude(x, emb_pad, bond_emb):
    return pl.pallas_call(
        _prelude_body,
        grid=(NBLK,),
        in_specs=[
            pl.BlockSpec((RB, 9), lambda i: (i, 0)),
            pl.BlockSpec((9, 128, D), lambda i: (0, 0, 0)),
            pl.BlockSpec((3, 16, D), lambda i: (0, 0, 0)),
        ],
        out_specs=[
            pl.BlockSpec((RB, D), lambda i: (i, 0)),
            pl.BlockSpec((4096, D), lambda i: (0, 0)),
        ],
        out_shape=[
            jax.ShapeDtypeStruct((N, D), jnp.float32),
            jax.ShapeDtypeStruct((4096, D), jnp.float32),
        ],
    )(x, emb_pad, bond_emb)


_sc_mesh = plsc.VectorSubcoreMesh(
    core_axis_name="c", subcore_axis_name="s", num_cores=NC, num_subcores=NS)


HS = 4
NCPAD = -(-NCHUNK // HS) * HS


@functools.partial(
    pl.kernel,
    out_type=jax.ShapeDtypeStruct((NC * N, D), jnp.float32),
    mesh=_sc_mesh,
    scratch_types=[
        [pltpu.VMEM((KCH,), jnp.int32) for _ in range(HS)],
        [pltpu.VMEM((KCH,), jnp.int32) for _ in range(HS)],
        [pltpu.VMEM((KCH,), jnp.int32) for _ in range(HS)],
        [pltpu.VMEM((KCH, D), jnp.float32) for _ in range(HS)],
        [pltpu.VMEM((KCH, D), jnp.float32) for _ in range(HS)],
        pltpu.VMEM((8, D), jnp.float32),
        pltpu.VMEM_SHARED((N, D), jnp.float32),
        [pltpu.SemaphoreType.DMA for _ in range(HS)],
        [pltpu.SemaphoreType.DMA for _ in range(HS)],
        [pltpu.SemaphoreType.DMA for _ in range(HS)],
        [pltpu.SemaphoreType.DMA for _ in range(HS)],
        [pltpu.SemaphoreType.DMA for _ in range(HS)],
    ],
)
def _edge_kernel(h_hbm, t_hbm, src_hbm, key_hbm, dst_hbm, out_hbm,
                 sidxb, kidxb, didxb, hbufs, tbufs, zbuf, aggr,
                 isem, dsem, gsh, gst, ssem):
    cid = lax.axis_index("c")
    sid = lax.axis_index("s")
    wid = sid * NC + cid
    ebase = wid * EPW

    zv = jnp.zeros((16,), jnp.float32)

    @pl.loop(0, 8)
    def _zero_zbuf(r):
        for c8 in range(D // 16):
            zbuf[r, pl.ds(c8 * 16, 16)] = zv

    @pl.loop(sid, NZCH, step=NS)
    def _zero_aggr(j):
        zcps = [pltpu.make_async_copy(zbuf, aggr.at[pl.ds(j * ZR + i * 8, 8)],
                                      isem[0]) for i in range(ZR // 8)]
        for cp in zcps:
            cp.start()
        for cp in zcps:
            cp.wait()

    plsc.subcore_barrier()

    def sk_loads(k, s):
        off = ebase + k * KCH
        return (pltpu.make_async_copy(src_hbm.at[pl.ds(off, KCH)],
                                      sidxb[s], isem[s]),
                pltpu.make_async_copy(key_hbm.at[pl.ds(off, KCH)],
                                      kidxb[s], isem[s]))

    def d_load(k, s):
        off = ebase + k * KCH
        return pltpu.make_async_copy(dst_hbm.at[pl.ds(off, KCH)],
                                     didxb[s], dsem[s])

    def h_gather(s):
        return pltpu.make_async_copy(h_hbm.at[sidxb[s]], hbufs[s], gsh[s])

    def t_gather(s):
        return pltpu.make_async_copy(t_hbm.at[kidxb[s]], tbufs[s], gst[s])

    def scatter(s):
        return pltpu.make_async_copy(hbufs[s], aggr.at[didxb[s]], ssem[s])

    for k in range(3):
        for cp in sk_loads(k, k):
            cp.start()
    for k in range(2):
        d_load(k, k).start()
    for k in range(2):
        for cp in sk_loads(k, k):
            cp.wait()
        h_gather(k).start()
        t_gather(k).start()

    @pl.loop(0, NCPAD, step=HS)
    def _chunks(ci):
        for j in range(HS):
            k = ci + j
            s = j
            s2 = (j + 2) % HS
            s3 = (j + 3) % HS

            @pl.when(k >= 2)
            def _(s2=s2):
                scatter(s2).wait()

            @pl.when(k + 2 < NCHUNK)
            def _(k=k, s2=s2):
                d_load(k + 2, s2).start()

            @pl.when(k + 3 < NCHUNK)
            def _(k=k, s3=s3):
                for cp in sk_loads(k + 3, s3):
                    cp.start()

            @pl.when(k + 2 < NCHUNK)
            def _(k=k, s2=s2):
                for cp in sk_loads(k + 2, s2):
                    cp.wait()
                h_gather(s2).start()
                t_gather(s2).start()

            @pl.when(k < NCHUNK)
            def _(k=k, s=s):
                h_gather(s).wait()
                t_gather(s).wait()
                hbuf, tbuf = hbufs[s], tbufs[s]

                @pl.loop(0, KCH)
                def _rows(r):
                    for c8 in range(D // 16):
                        sl = pl.ds(c8 * 16, 16)
                        hbuf[r, sl] = jnp.maximum(hbuf[r, sl] + tbuf[r, sl],
                                                  0.0)

                d_load(k, s).wait()
                scatter(s).start(add=True)

    plsc.subcore_barrier()

    def wb_copy(j):
        return pltpu.make_async_copy(aggr.at[pl.ds(j * ZR, ZR)],
                                     out_hbm.at[pl.ds(cid * N + j * ZR, ZR)],
                                     dsem[0])

    @pl.loop(sid, NZCH, step=NS)
    def _writeback(j):
        wb_copy(j).start()

    @pl.loop(sid, NZCH, step=NS)
    def _writeback_drain(j):
        wb_copy(j).wait()


def _update_body(h_ref, a0_ref, a1_ref, wa_ref, ba_ref, wb_ref, bb_ref, o_ref):
    z = h_ref[...] + a0_ref[0] + a1_ref[0]
    y = jnp.maximum(
        jnp.dot(z, wa_ref[...], preferred_element_type=jnp.float32)
        + ba_ref[...], 0.0)
    o_ref[...] = (jnp.dot(y, wb_ref[...], preferred_element_type=jnp.float32)
                  + bb_ref[...])


def _node_update(h, agg, wa, ba, wb, bb):
    return pl.pallas_call(
        _update_body,
        grid=(NBLK,),
        in_specs=[
            pl.BlockSpec((RB, D), lambda i: (i, 0)),
            pl.BlockSpec((1, RB, D), lambda i: (0, i, 0)),
            pl.BlockSpec((1, RB, D), lambda i: (1, i, 0)),
            pl.BlockSpec((D, D), lambda i: (0, 0)),
            pl.BlockSpec((1, D), lambda i: (0, 0)),
            pl.BlockSpec((D, D), lambda i: (0, 0)),
            pl.BlockSpec((1, D), lambda i: (0, 0)),
        ],
        out_specs=pl.BlockSpec((RB, D), lambda i: (i, 0)),
        out_shape=jax.ShapeDtypeStruct((N, D), jnp.float32),
    )(h, agg, agg, wa, ba, wb, bb)


def _update_pool_body(h_ref, a0_ref, a1_ref, wa_ref, ba_ref, wb_ref, bb_ref,
                      bt_ref, rd_ref, w1a_ref, w1b_ref, b1_ref,
                      w2_ref, b2_ref, w3_ref, b3_ref, o_ref, sums, cnts):
    i = pl.program_id(0)

    z = h_ref[...] + a0_ref[0] + a1_ref[0]
    y = jnp.maximum(
        jnp.dot(z, wa_ref[...], preferred_element_type=jnp.float32)
        + ba_ref[...], 0.0)
    h2 = (jnp.dot(y, wb_ref[...], preferred_element_type=jnp.float32)
          + bb_ref[...])

    @pl.when(i == 0)
    def _():
        sums[...] = jnp.zeros((G, D), jnp.float32)
        cnts[...] = jnp.zeros((G, D), jnp.float32)

    b = bt_ref[0]
    oh = (lax.broadcasted_iota(jnp.int32, (G, RB), 0)
          == jnp.broadcast_to(b, (G, RB))).astype(jnp.float32)
    sums[...] += jnp.dot(oh, h2, preferred_element_type=jnp.float32)
    cnts[...] += jnp.dot(oh, jnp.ones((RB, D), jnp.float32),
                         preferred_element_type=jnp.float32)

    @pl.when(i == NBLK - 1)
    def _():
        pooled = sums[...] / jnp.maximum(cnts[...], 1.0)
        z1 = jnp.maximum(
            jnp.dot(pooled, w1a_ref[...], preferred_element_type=jnp.float32)
            + jnp.dot(rd_ref[...], w1b_ref[...],
                      preferred_element_type=jnp.float32)
            + b1_ref[...], 0.0)
        z2 = jnp.maximum(
            jnp.dot(z1, w2_ref[...], preferred_element_type=jnp.float32)
            + b2_ref[...], 0.0)
        o_ref[...] = (jnp.dot(z2, w3_ref[...],
                              preferred_element_type=jnp.float32)
                      + b3_ref[...])


def _update_pool(h, agg, wa, ba, wb, bb, batch3, rdkit,
                 w1a, w1b, b1, w2, b2, w3p, b3p):
    return pl.pallas_call(
        _update_pool_body,
        grid=(NBLK,),
        in_specs=[
            pl.BlockSpec((RB, D), lambda i: (i, 0)),
            pl.BlockSpec((1, RB, D), lambda i: (0, i, 0)),
            pl.BlockSpec((1, RB, D), lambda i: (1, i, 0)),
            pl.BlockSpec((D, D), lambda i: (0, 0)),
            pl.BlockSpec((1, D), lambda i: (0, 0)),
            pl.BlockSpec((D, D), lambda i: (0, 0)),
            pl.BlockSpec((1, D), lambda i: (0, 0)),
            pl.BlockSpec((1, 1, RB), lambda i: (i, 0, 0)),
            pl.BlockSpec((G, RDKIT), lambda i: (0, 0)),
            pl.BlockSpec((D, HID), lambda i: (0, 0)),
            pl.BlockSpec((RDKIT, HID), lambda i: (0, 0)),
            pl.BlockSpec((1, HID), lambda i: (0, 0)),
            pl.BlockSpec((HID, HID // 2), lambda i: (0, 0)),
            pl.BlockSpec((1, HID // 2), lambda i: (0, 0)),
            pl.BlockSpec((HID // 2, 128), lambda i: (0, 0)),
            pl.BlockSpec((1, 128), lambda i: (0, 0)),
        ],
        out_specs=pl.BlockSpec((G, 128), lambda i: (0, 0)),
        out_shape=jax.ShapeDtypeStruct((G, 128), jnp.float32),
        scratch_shapes=[
            pltpu.VMEM((G, D), jnp.float32),
            pltpu.VMEM((G, D), jnp.float32),
        ],
    )(h, agg, agg, wa, ba, wb, bb, batch3, rdkit,
      w1a, w1b, b1, w2, b2, w3p, b3p)


def kernel(x, edge_index, edge_attr, batch, rdkit_feats, atom_emb, bond_emb,
           W1a, b1a, W1b, b1b, W2a, b2a, W2b, b2b,
           M1W, M1b, M2W, M2b, M3W, M3b):
    emb_pad = jnp.pad(atom_emb, ((0, 0), (0, 128 - 100), (0, 0)))
    h, ttab = _prelude(x, emb_pad, bond_emb)

    src = edge_index[0]
    dst = edge_index[1]
    key = edge_attr[:, 0] * 256 + edge_attr[:, 1] * 16 + edge_attr[:, 2]

    agg = _edge_kernel(h, ttab, src, key, dst).reshape(2, N, D)
    h = _node_update(h, agg, W1a, b1a[None], W1b, b1b[None])
    agg = _edge_kernel(h, ttab, src, key, dst).reshape(2, N, D)

    batch3 = batch.reshape(NBLK, 1, RB)
    w3p = jnp.pad(M3W, ((0, 0), (0, 127)))
    b3p = jnp.pad(M3b[None], ((0, 0), (0, 127)))
    out_full = _update_pool(h, agg, W2a, b2a[None], W2b, b2b[None],
                            batch3, rdkit_feats,
                            M1W[:D], M1W[D:], M1b[None], M2W, M2b[None],
                            w3p, b3p)
    return out_full[:, :1]
